# Initial kernel scaffold; baseline (speedup 1.0000x reference)
import jax
import jax.numpy as jnp
from jax.experimental import pallas as pl


def kernel(x, sWihf, sWhhf, sbf, sWihb, sWhhb, sbb, dWihf, dWhhf, dbf, dWihb, dWhhb, dbb, fcW, fcb, recover_idx, num_sent_per_document, sent_lengths):
    raise NotImplementedError("write your pallas kernel here")


# speedup vs baseline: 27.1630x; 27.1630x over previous
import functools

import jax
import jax.numpy as jnp
from jax.experimental import pallas as pl
from jax.experimental.pallas import tpu as pltpu

NS, T, E, H = 512, 64, 300, 256


def _sigmoid(x):
    return jax.nn.sigmoid(x)


def _lstm_update(xt, h, c, Wih_ref, Whh_ref, b_ref):
    gates = (
        jnp.dot(xt, Wih_ref[...], preferred_element_type=jnp.float32)
        + jnp.dot(h, Whh_ref[...], preferred_element_type=jnp.float32)
        + b_ref[...]
    )
    Hh = Whh_ref.shape[0]
    i = _sigmoid(gates[:, :Hh])
    f = _sigmoid(gates[:, Hh:2 * Hh])
    g = jnp.tanh(gates[:, 2 * Hh:3 * Hh])
    o = _sigmoid(gates[:, 3 * Hh:])
    c_new = f * c + i * g
    h_new = o * jnp.tanh(c_new)
    return h_new, c_new


def _sent_kernel(lens_ref, xf_ref, xb_ref, Wf_ref, Uf_ref, bf_ref,
                 Wb_ref, Ub_ref, bb_ref, out_ref, hf, cf, hb, cb):
    t = pl.program_id(0)

    @pl.when(t == 0)
    def _init():
        hf[...] = jnp.zeros_like(hf)
        cf[...] = jnp.zeros_like(cf)
        hb[...] = jnp.zeros_like(hb)
        cb[...] = jnp.zeros_like(cb)

    lens = lens_ref[...]

    h_new, c_new = _lstm_update(xf_ref[0], hf[...], cf[...], Wf_ref, Uf_ref, bf_ref)
    m = t < lens
    hf[...] = jnp.where(m, h_new, hf[...])
    cf[...] = jnp.where(m, c_new, cf[...])

    h_new, c_new = _lstm_update(xb_ref[0], hb[...], cb[...], Wb_ref, Ub_ref, bb_ref)
    mb = (T - 1 - t) < lens
    hb[...] = jnp.where(mb, h_new, hb[...])
    cb[...] = jnp.where(mb, c_new, cb[...])

    @pl.when(t == T - 1)
    def _emit():
        out_ref[:, :H] = hf[...]
        out_ref[:, H:] = hb[...]


def _doc_kernel(ridx_ref, offs_ref, maxc_ref, cnts_ref, semb_ref,
                Wf_ref, Uf_ref, bf_ref, Wb_ref, Ub_ref, bb_ref,
                fcW_ref, fcb_ref, out_ref, xtf, xtb, hf, cf, hb, cb):
    B = cnts_ref.shape[0]
    maxc = maxc_ref[0]
    cnts = cnts_ref[...]

    hf[...] = jnp.zeros_like(hf)
    cf[...] = jnp.zeros_like(cf)
    hb[...] = jnp.zeros_like(hb)
    cb[...] = jnp.zeros_like(cb)

    def gather(t, dst):
        for d in range(B):
            addr = jnp.minimum(offs_ref[d] + t, NS - 1)
            j = ridx_ref[addr]
            dst[d, :] = semb_ref[j, :]

    def body(s, carry):
        gather(s, xtf)
        h_new, c_new = _lstm_update(xtf[...], hf[...], cf[...], Wf_ref, Uf_ref, bf_ref)
        m = s < cnts
        hf[...] = jnp.where(m, h_new, hf[...])
        cf[...] = jnp.where(m, c_new, cf[...])
        tb = maxc - 1 - s
        gather(tb, xtb)
        h_new, c_new = _lstm_update(xtb[...], hb[...], cb[...], Wb_ref, Ub_ref, bb_ref)
        mb = tb < cnts
        hb[...] = jnp.where(mb, h_new, hb[...])
        cb[...] = jnp.where(mb, c_new, cb[...])
        return carry

    jax.lax.fori_loop(0, maxc, body, 0, unroll=False)

    logits = (
        jnp.dot(hf[...], fcW_ref[:H, :], preferred_element_type=jnp.float32)
        + jnp.dot(hb[...], fcW_ref[H:, :], preferred_element_type=jnp.float32)
        + fcb_ref[0, 0]
    )
    out_ref[...] = _sigmoid(logits)


@jax.jit
def kernel(x, sWihf, sWhhf, sbf, sWihb, sWhhb, sbb, dWihf, dWhhf, dbf,
           dWihb, dWhhb, dbb, fcW, fcb, recover_idx, num_sent_per_document,
           sent_lengths):
    B = num_sent_per_document.shape[0]
    x_tm = jnp.transpose(x, (1, 0, 2))
    lens2d = sent_lengths.reshape(NS, 1)

    sent_emb_sorted = pl.pallas_call(
        _sent_kernel,
        grid=(T,),
        in_specs=[
            pl.BlockSpec((NS, 1), lambda t: (0, 0)),
            pl.BlockSpec((1, NS, E), lambda t: (t, 0, 0)),
            pl.BlockSpec((1, NS, E), lambda t: (T - 1 - t, 0, 0)),
            pl.BlockSpec((E, 4 * H), lambda t: (0, 0)),
            pl.BlockSpec((H, 4 * H), lambda t: (0, 0)),
            pl.BlockSpec((1, 4 * H), lambda t: (0, 0)),
            pl.BlockSpec((E, 4 * H), lambda t: (0, 0)),
            pl.BlockSpec((H, 4 * H), lambda t: (0, 0)),
            pl.BlockSpec((1, 4 * H), lambda t: (0, 0)),
        ],
        out_specs=pl.BlockSpec((NS, 2 * H), lambda t: (0, 0)),
        out_shape=jax.ShapeDtypeStruct((NS, 2 * H), jnp.float32),
        scratch_shapes=[pltpu.VMEM((NS, H), jnp.float32)] * 4,
    )(lens2d, x_tm, x_tm, sWihf, sWhhf, sbf.reshape(1, -1),
      sWihb, sWhhb, sbb.reshape(1, -1))

    counts = num_sent_per_document.astype(jnp.int32)
    offsets = jnp.concatenate(
        [jnp.zeros((1,), jnp.int32), jnp.cumsum(counts)[:-1]])
    maxc = jnp.max(counts).reshape(1)

    out2d = pl.pallas_call(
        _doc_kernel,
        in_specs=[
            pl.BlockSpec(memory_space=pltpu.SMEM),
            pl.BlockSpec(memory_space=pltpu.SMEM),
            pl.BlockSpec(memory_space=pltpu.SMEM),
            pl.BlockSpec((B, 1), lambda: (0, 0)),
            pl.BlockSpec((NS, 2 * H), lambda: (0, 0)),
            pl.BlockSpec((2 * H, 4 * H), lambda: (0, 0)),
            pl.BlockSpec((H, 4 * H), lambda: (0, 0)),
            pl.BlockSpec((1, 4 * H), lambda: (0, 0)),
            pl.BlockSpec((2 * H, 4 * H), lambda: (0, 0)),
            pl.BlockSpec((H, 4 * H), lambda: (0, 0)),
            pl.BlockSpec((1, 4 * H), lambda: (0, 0)),
            pl.BlockSpec((2 * H, 1), lambda: (0, 0)),
            pl.BlockSpec((1, 1), lambda: (0, 0)),
        ],
        out_specs=pl.BlockSpec((B, 1), lambda: (0, 0)),
        out_shape=jax.ShapeDtypeStruct((B, 1), jnp.float32),
        scratch_shapes=[pltpu.VMEM((B, 2 * H), jnp.float32)] * 2
        + [pltpu.VMEM((B, H), jnp.float32)] * 4,
    )(recover_idx.astype(jnp.int32), offsets, maxc, counts.reshape(B, 1),
      sent_emb_sorted, dWihf, dWhhf, dbf.reshape(1, -1),
      dWihb, dWhhb, dbb.reshape(1, -1), fcW, fcb.reshape(1, 1))

    return out2d.reshape(-1)
